# baseline (speedup 1.0000x reference)
import jax
import jax.numpy as jnp
from jax import lax
from jax.experimental import pallas as pl
from jax.experimental.pallas import tpu as pltpu
from jax.experimental.pallas import tpu_sc as plsc

N = 10000
EMB = 256
E = 160000
NC = 2
NS = 16
L = 16

N2 = 10240
RT = 640
HALF = 5120
AGGR = 5128
TRASH = 5120
SENT = 10240
HROWS = 10248
FT = 320

EC = E // NS
SE = 2048
NSTRIP = 5
CH = 32
NCH = SE // CH
C2 = 32
C3 = 16
C5 = 64


def _rsqrt_newton(x):
    i = lax.bitcast_convert_type(x, jnp.int32)
    i = jnp.full_like(i, 0x5F3759DF) - lax.shift_right_arithmetic(
        i, jnp.ones_like(i))
    y = lax.bitcast_convert_type(i, jnp.float32)
    half_x = x * 0.5
    for _ in range(4):
        y = y * (1.5 - half_x * y * y)
    return y


def _sc_body(rows_hbm, cols_hbm, h_hbm, bias_hbm,
             item_hbm, hs_hbm,
             stage, eidx, gbuf, zbuf, dinv_t, dinv_f, biasv,
             hist_s, dinv_s, agg_s,
             sem0, sem1):
    cid = lax.axis_index("c")
    sid = lax.axis_index("s")
    base = cid * HALF
    e0 = sid * EC
    iota = lax.iota(jnp.int32, L)

    pltpu.sync_copy(bias_hbm, biasv)
    scope = jax.named_scope

    def _fill_zbuf(val):
        def _row(i, carry):
            zbuf[i, pl.ds(0, L)] = jnp.full((L,), val, jnp.float32)
            return carry
        lax.fori_loop(0, C2, _row, 0)

    _fill_zbuf(0.0)
    for j in range(RT // C2):
        pltpu.sync_copy(zbuf, hist_s.at[pl.ds(sid * RT + j * C2, C2)])

    @pl.when(sid == 0)
    def _zero_hist_tail():
        pltpu.sync_copy(zbuf.at[pl.ds(0, HROWS - N2)],
                        hist_s.at[pl.ds(N2, HROWS - N2)])

    plsc.subcore_barrier()
    _fill_zbuf(1.0)
    p1 = scope("p1_hist"); p1.__enter__()

    for s in range(NSTRIP):
        nreal = min(EC - s * SE, SE)
        pltpu.sync_copy(rows_hbm.at[pl.ds(e0 + s * SE, nreal)],
                        stage.at[pl.ds(0, nreal)])
        for i in range((SE - nreal) // L):
            stage[pl.ds(nreal + i * L, L)] = jnp.full((L,), SENT, jnp.int32)

        def _xform_raw(j, carry):
            for k in range(CH // L):
                eidx[j, pl.ds(k * L, L)] = stage[pl.ds(j * CH + k * L, L)]
            return carry
        lax.fori_loop(0, NCH, _xform_raw, 0)

        def _fire(j, carry):
            pltpu.async_copy(zbuf, hist_s.at[eidx.at[j]], sem0, add=True)
            return carry
        lax.fori_loop(0, NCH, _fire, 0)

        def _drain(j, carry):
            pltpu.make_async_copy(zbuf, hist_s.at[eidx.at[0]], sem0).wait()
            return carry
        lax.fori_loop(0, NCH, _drain, 0)

    p1.__exit__(None, None, None)
    plsc.subcore_barrier()
    p2 = scope("p2_dinv"); p2.__enter__()

    def _deg_chunk(c, carry):
        pltpu.sync_copy(hist_s.at[pl.ds(sid * RT + c * C2, C2)], zbuf)

        def _deg_group(g, carry2):
            acc = jnp.zeros((L,), jnp.float32)
            for i in range(L):
                acc = jnp.where(iota == i, zbuf[g * L + i, pl.ds(0, L)], acc)
            dinv_t[pl.ds(c * C2 + g * L, L)] = _rsqrt_newton(acc + 1.0)
            return carry2
        lax.fori_loop(0, C2 // L, _deg_group, 0)
        return carry
    lax.fori_loop(0, RT // C2, _deg_chunk, 0)

    pltpu.sync_copy(dinv_t, dinv_s.at[pl.ds(sid * RT, RT)])
    p2.__exit__(None, None, None)
    plsc.subcore_barrier()
    p3 = scope("p3_hs"); p3.__enter__()

    nhs = jnp.minimum(N - sid * RT, RT) // C3

    def _hs_chunk(c, carry):
        r0 = sid * RT + c * C3
        pltpu.sync_copy(h_hbm.at[pl.ds(r0, C3)], gbuf.at[pl.ds(0, C3)])
        dvs = dinv_t[pl.ds(c * C3, L)]
        for i in range(L):
            dv = jnp.full((L,), dvs[i])
            for k in range(EMB // L):
                gbuf[i, k // 8, pl.ds((k % 8) * L, L)] = (
                    gbuf[i, k // 8, pl.ds((k % 8) * L, L)] * dv)
        pltpu.sync_copy(gbuf.at[pl.ds(0, C3)], hs_hbm.at[pl.ds(r0, C3)])

        @pl.when((r0 >= base) & (r0 + C3 <= jnp.minimum(base + HALF, N)))
        def _agg_init():
            pltpu.sync_copy(gbuf.at[pl.ds(0, C3)],
                            agg_s.at[pl.ds(r0 - base, C3)])
        return carry
    lax.fori_loop(0, nhs, _hs_chunk, 0)

    @pl.when(sid == 0)
    def _zero_agg_pad():
        def _zero_grow(rr, carry):
            for k in range(EMB // L):
                gbuf[rr, k // 8, pl.ds((k % 8) * L, L)] = (
                    jnp.zeros((L,), jnp.float32))
            return carry
        lax.fori_loop(0, C5, _zero_grow, 0)

        @pl.when(cid == 0)
        def _c0():
            pltpu.sync_copy(gbuf.at[pl.ds(0, AGGR - HALF)],
                            agg_s.at[pl.ds(HALF, AGGR - HALF)])

        @pl.when(cid == 1)
        def _c1():
            for t in range((HALF - (N - HALF)) // C5):
                pltpu.sync_copy(gbuf.at[pl.ds(0, C5)],
                                agg_s.at[pl.ds(N - HALF + t * C5, C5)])
            pltpu.sync_copy(gbuf.at[pl.ds(0, AGGR - HALF)],
                            agg_s.at[pl.ds(HALF, AGGR - HALF)])

    p3.__exit__(None, None, None)
    plsc.subcore_barrier()
    p4 = scope("p4_edges"); p4.__enter__()

    b0 = gbuf.at[pl.ds(0, CH)]
    b1 = gbuf.at[pl.ds(CH, CH)]
    for s in range(NSTRIP):
        nreal = min(EC - s * SE, SE)
        pltpu.sync_copy(rows_hbm.at[pl.ds(e0 + s * SE, nreal)],
                        stage.at[pl.ds(0, nreal)])
        for i in range((SE - nreal) // L):
            stage[pl.ds(nreal + i * L, L)] = jnp.full((L,), SENT, jnp.int32)

        def _xform_loc(j, carry):
            for k in range(CH // L):
                v = stage[pl.ds(j * CH + k * L, L)]
                in_half = (v >= base) & (v < base + HALF)
                eidx[j, pl.ds(k * L, L)] = jnp.where(in_half, v - base, TRASH)
            return carry
        lax.fori_loop(0, NCH, _xform_loc, 0)

        pltpu.sync_copy(cols_hbm.at[pl.ds(e0 + s * SE, nreal)],
                        stage.at[pl.ds(0, nreal)])
        for i in range((SE - nreal) // L):
            stage[pl.ds(nreal + i * L, L)] = jnp.zeros((L,), jnp.int32)

        pltpu.async_copy(hs_hbm.at[stage.at[pl.ds(0, CH)]], b0, sem0)
        pltpu.async_copy(hs_hbm.at[stage.at[pl.ds(CH, CH)]], b1, sem1)

        def _edge_pair(j2, carry):
            j = j2 * 2
            pltpu.make_async_copy(hs_hbm.at[pl.ds(0, CH)], b0, sem0).wait()
            pltpu.sync_copy(b0, agg_s.at[eidx.at[j]], add=True)

            @pl.when(j + 2 < NCH)
            def _pf0():
                pltpu.async_copy(hs_hbm.at[stage.at[pl.ds((j + 2) * CH, CH)]],
                                 b0, sem0)

            pltpu.make_async_copy(hs_hbm.at[pl.ds(0, CH)], b1, sem1).wait()
            pltpu.sync_copy(b1, agg_s.at[eidx.at[j + 1]], add=True)

            @pl.when(j + 3 < NCH)
            def _pf1():
                pltpu.async_copy(hs_hbm.at[stage.at[pl.ds((j + 3) * CH, CH)]],
                                 b1, sem1)
            return carry
        lax.fori_loop(0, NCH // 2, _edge_pair, 0)

    p4.__exit__(None, None, None)
    plsc.subcore_barrier()
    p5 = scope("p5_fin"); p5.__enter__()

    real = jnp.minimum(base + HALF, N) - base
    pltpu.sync_copy(dinv_s.at[pl.ds(base + sid * FT, FT)], dinv_f)
    def _fin_chunk(c, carry):
        start = jnp.minimum(sid * FT + c * C5, real - C5)
        loff = start - sid * FT
        pltpu.sync_copy(agg_s.at[pl.ds(start, C5)], gbuf.at[pl.ds(0, C5)])

        def _fin_group(g, carry2):
            dvs = dinv_f[pl.ds(loff + g * L, L)]
            for i in range(L):
                dv = jnp.full((L,), dvs[i])
                row = g * L + i
                for k in range(EMB // L):
                    v = (gbuf[row, k // 8, pl.ds((k % 8) * L, L)] * dv
                         + biasv[pl.ds(k * L, L)])
                    gbuf[row, k // 8, pl.ds((k % 8) * L, L)] = jnp.maximum(
                        v, 0.0)
            return carry2
        lax.fori_loop(0, C5 // L, _fin_group, 0)
        pltpu.sync_copy(gbuf.at[pl.ds(0, C5)],
                        item_hbm.at[pl.ds(base + start, C5)])
        return carry
    lax.fori_loop(0, FT // C5, _fin_chunk, 0)
    p5.__exit__(None, None, None)


@jax.jit
def _igcn_sc(rows, cols, h, bias):
    mesh = plsc.VectorSubcoreMesh(core_axis_name="c", subcore_axis_name="s",
                                  num_cores=NC, num_subcores=NS)
    f = pl.kernel(
        _sc_body,
        out_type=(
            jax.ShapeDtypeStruct((N, 2, 128), jnp.float32),
            jax.ShapeDtypeStruct((N, 2, 128), jnp.float32),
        ),
        mesh=mesh,
        scratch_types=[
            pltpu.VMEM((SE,), jnp.int32),
            pltpu.VMEM((NCH, CH), jnp.int32),
            pltpu.VMEM((2 * CH, 2, 128), jnp.float32),
            pltpu.VMEM((C2, L), jnp.float32),
            pltpu.VMEM((RT,), jnp.float32),
            pltpu.VMEM((FT,), jnp.float32),
            pltpu.VMEM((EMB,), jnp.float32),
            pltpu.VMEM_SHARED((HROWS, L), jnp.float32),
            pltpu.VMEM_SHARED((N2,), jnp.float32),
            pltpu.VMEM_SHARED((AGGR, 2, 128), jnp.float32),
            pltpu.SemaphoreType.DMA,
            pltpu.SemaphoreType.DMA,
        ],
    )
    item_h, _hs = f(rows, cols, h, bias)
    return item_h


def kernel(edge_index, user_embeddings, gcn_kernel, gcn_bias):
    rows = edge_index[0].astype(jnp.int32)
    cols = edge_index[1].astype(jnp.int32)
    h3 = gcn_kernel.reshape(N, 2, 128)
    item_h = _igcn_sc(rows, cols, h3, gcn_bias)
    return (user_embeddings, item_h.reshape(N, EMB))


# speedup vs baseline: 1.5817x; 1.5817x over previous
import jax
import jax.numpy as jnp
from jax import lax
from jax.experimental import pallas as pl
from jax.experimental.pallas import tpu as pltpu
from jax.experimental.pallas import tpu_sc as plsc

N = 10000
EMB = 256
E = 160000
NC = 2
NS = 16
L = 16
W = 128

N2 = 10240
RT = 640
AGGR = 10248
TRASH = 10240
HROWS = 10248
SENT = 10240

EC = E // NS
SE = 2048
NSTRIP = 5
CH = 64
NCH = SE // CH
CHH = 32
NCHH = SE // CHH
C2 = 32
C3 = 16
C5 = 64


def _rsqrt_newton(x):
    i = lax.bitcast_convert_type(x, jnp.int32)
    i = jnp.full_like(i, 0x5F3759DF) - lax.shift_right_arithmetic(
        i, jnp.ones_like(i))
    y = lax.bitcast_convert_type(i, jnp.float32)
    half_x = x * 0.5
    for _ in range(4):
        y = y * (1.5 - half_x * y * y)
    return y


def _sc_body(rows_hbm, cols_hbm, h_hbm, bias_hbm,
             item_hbm, hs_hbm,
             rstage, cstage, gbuf, zbuf, dinv_t, biasv,
             hist_s, agg_s,
             sem0, sem1):
    cid = lax.axis_index("c")
    sid = lax.axis_index("s")
    e0 = sid * EC
    r0t = sid * RT
    hs0 = cid * N2
    iota = lax.iota(jnp.int32, L)

    pltpu.sync_copy(bias_hbm.at[pl.ds(cid * W, W)], biasv)
    scope = jax.named_scope

    def _fill_zbuf(val):
        def _row(i, carry):
            zbuf[i, pl.ds(0, L)] = jnp.full((L,), val, jnp.float32)
            return carry
        lax.fori_loop(0, C2, _row, 0)

    _fill_zbuf(0.0)
    for j in range(RT // C2):
        pltpu.sync_copy(zbuf, hist_s.at[pl.ds(r0t + j * C2, C2)])

    @pl.when(sid == 0)
    def _zero_hist_tail():
        pltpu.sync_copy(zbuf.at[pl.ds(0, HROWS - N2)],
                        hist_s.at[pl.ds(N2, HROWS - N2)])

    plsc.subcore_barrier()
    _fill_zbuf(1.0)
    p1 = scope("p1_hist"); p1.__enter__()

    for s in range(NSTRIP):
        nreal = min(EC - s * SE, SE)
        pltpu.sync_copy(rows_hbm.at[pl.ds(e0 + s * SE, nreal)],
                        rstage.at[pl.ds(0, nreal)])
        for i in range((SE - nreal) // L):
            rstage[pl.ds(nreal + i * L, L)] = jnp.full((L,), SENT, jnp.int32)

        def _fire(j, carry):
            pltpu.async_copy(zbuf, hist_s.at[rstage.at[pl.ds(j * CHH, CHH)]],
                             sem0, add=True)
            return carry
        lax.fori_loop(0, NCHH, _fire, 0)

        def _drain(j, carry):
            pltpu.make_async_copy(
                zbuf, hist_s.at[rstage.at[pl.ds(0, CHH)]], sem0).wait()
            return carry
        lax.fori_loop(0, NCHH, _drain, 0)

    p1.__exit__(None, None, None)
    plsc.subcore_barrier()
    p2 = scope("p2_dinv"); p2.__enter__()

    def _deg_chunk(c, carry):
        pltpu.sync_copy(hist_s.at[pl.ds(r0t + c * C2, C2)], zbuf)

        def _deg_group(g, carry2):
            acc = jnp.zeros((L,), jnp.float32)
            for i in range(L):
                acc = jnp.where(iota == i, zbuf[g * L + i, pl.ds(0, L)], acc)
            dinv_t[pl.ds(c * C2 + g * L, L)] = _rsqrt_newton(acc + 1.0)
            return carry2
        lax.fori_loop(0, C2 // L, _deg_group, 0)
        return carry
    lax.fori_loop(0, RT // C2, _deg_chunk, 0)

    p2.__exit__(None, None, None)
    p3 = scope("p3_hs"); p3.__enter__()

    nhs = jnp.minimum(N - r0t, RT) // C3

    def _hs_chunk(c, carry):
        r0 = r0t + c * C3
        pltpu.sync_copy(h_hbm.at[pl.ds(cid * N + r0, C3)], gbuf.at[pl.ds(0, C3)])
        dvs = dinv_t[pl.ds(c * C3, L)]
        for i in range(L):
            dv = jnp.full((L,), dvs[i])
            for k in range(W // L):
                gbuf[i, pl.ds(k * L, L)] = gbuf[i, pl.ds(k * L, L)] * dv
        pltpu.sync_copy(gbuf.at[pl.ds(0, C3)],
                        hs_hbm.at[pl.ds(hs0 + r0, C3)])
        pltpu.sync_copy(gbuf.at[pl.ds(0, C3)], agg_s.at[pl.ds(r0, C3)])
        return carry
    lax.fori_loop(0, nhs, _hs_chunk, 0)

    p3.__exit__(None, None, None)
    plsc.subcore_barrier()
    p4 = scope("p4_edges"); p4.__enter__()

    b0 = gbuf.at[pl.ds(0, CH)]
    b1 = gbuf.at[pl.ds(CH, CH)]
    for s in range(NSTRIP):
        nreal = min(EC - s * SE, SE)
        pltpu.sync_copy(rows_hbm.at[pl.ds(e0 + s * SE, nreal)],
                        rstage.at[pl.ds(0, nreal)])
        pltpu.sync_copy(cols_hbm.at[pl.ds(e0 + s * SE, nreal)],
                        cstage.at[pl.ds(0, nreal)])
        for i in range((SE - nreal) // L):
            rstage[pl.ds(nreal + i * L, L)] = jnp.full((L,), TRASH, jnp.int32)
            cstage[pl.ds(nreal + i * L, L)] = jnp.zeros((L,), jnp.int32)

        def _coff(j, carry):
            cstage[pl.ds(j * L, L)] = cstage[pl.ds(j * L, L)] + hs0
            return carry
        lax.fori_loop(0, SE // L, _coff, 0)

        pltpu.async_copy(hs_hbm.at[cstage.at[pl.ds(0, CH)]], b0, sem0)
        pltpu.async_copy(hs_hbm.at[cstage.at[pl.ds(CH, CH)]], b1, sem1)

        def _edge_pair(j2, carry):
            j = j2 * 2
            pltpu.make_async_copy(hs_hbm.at[pl.ds(0, CH)], b0, sem0).wait()
            pltpu.sync_copy(b0, agg_s.at[rstage.at[pl.ds(j * CH, CH)]],
                            add=True)

            @pl.when(j + 2 < NCH)
            def _pf0():
                pltpu.async_copy(hs_hbm.at[cstage.at[pl.ds((j + 2) * CH, CH)]],
                                 b0, sem0)

            pltpu.make_async_copy(hs_hbm.at[pl.ds(0, CH)], b1, sem1).wait()
            pltpu.sync_copy(b1, agg_s.at[rstage.at[pl.ds((j + 1) * CH, CH)]],
                            add=True)

            @pl.when(j + 3 < NCH)
            def _pf1():
                pltpu.async_copy(hs_hbm.at[cstage.at[pl.ds((j + 3) * CH, CH)]],
                                 b1, sem1)
            return carry
        lax.fori_loop(0, NCH // 2, _edge_pair, 0)

    p4.__exit__(None, None, None)
    plsc.subcore_barrier()
    p5 = scope("p5_fin"); p5.__enter__()

    real = jnp.minimum(N - r0t, RT)

    def _fin_chunk(c, carry):
        loff = jnp.minimum(c * C5, real - C5)
        pltpu.sync_copy(agg_s.at[pl.ds(r0t + loff, C5)], gbuf.at[pl.ds(0, C5)])

        def _fin_group(g, carry2):
            dvs = dinv_t[pl.ds(loff + g * L, L)]
            for i in range(L):
                dv = jnp.full((L,), dvs[i])
                row = g * L + i
                for k in range(W // L):
                    v = gbuf[row, pl.ds(k * L, L)] * dv + biasv[pl.ds(k * L, L)]
                    gbuf[row, pl.ds(k * L, L)] = jnp.maximum(v, 0.0)
            return carry2
        lax.fori_loop(0, C5 // L, _fin_group, 0)
        pltpu.sync_copy(gbuf.at[pl.ds(0, C5)],
                        item_hbm.at[pl.ds(cid * N + r0t + loff, C5)])
        return carry
    nfin = (real + C5 - 1) // C5
    lax.fori_loop(0, nfin, _fin_chunk, 0)
    p5.__exit__(None, None, None)


@jax.jit
def _igcn_sc(rows, cols, h, bias):
    mesh = plsc.VectorSubcoreMesh(core_axis_name="c", subcore_axis_name="s",
                                  num_cores=NC, num_subcores=NS)
    f = pl.kernel(
        _sc_body,
        out_type=(
            jax.ShapeDtypeStruct((NC * N, W), jnp.float32),
            jax.ShapeDtypeStruct((NC * N2, W), jnp.float32),
        ),
        mesh=mesh,
        scratch_types=[
            pltpu.VMEM((SE,), jnp.int32),
            pltpu.VMEM((SE,), jnp.int32),
            pltpu.VMEM((2 * CH, W), jnp.float32),
            pltpu.VMEM((C2, L), jnp.float32),
            pltpu.VMEM((RT,), jnp.float32),
            pltpu.VMEM((W,), jnp.float32),
            pltpu.VMEM_SHARED((HROWS, L), jnp.float32),
            pltpu.VMEM_SHARED((AGGR, W), jnp.float32),
            pltpu.SemaphoreType.DMA,
            pltpu.SemaphoreType.DMA,
        ],
    )
    item_h, _hs = f(rows, cols, h, bias)
    return item_h


def kernel(edge_index, user_embeddings, gcn_kernel, gcn_bias):
    rows = edge_index[0].astype(jnp.int32)
    cols = edge_index[1].astype(jnp.int32)
    h2 = gcn_kernel.reshape(N, NC, W).transpose(1, 0, 2).reshape(NC * N, W)
    item_flat = _igcn_sc(rows, cols, h2, gcn_bias)
    item_h = item_flat.reshape(NC, N, W).transpose(1, 0, 2).reshape(N, EMB)
    return (user_embeddings, item_h)


# speedup vs baseline: 1.5979x; 1.0102x over previous
import jax
import jax.numpy as jnp
from jax import lax
from jax.experimental import pallas as pl
from jax.experimental.pallas import tpu as pltpu
from jax.experimental.pallas import tpu_sc as plsc

N = 10000
EMB = 256
E = 160000
NC = 2
NS = 16
L = 16
W = 128

N2 = 10240
RT = 640
AGGR = 10008
TRASH = 10000
HROWS = 10008
SENT = 10000

EC = E // NS
SE = 1024
NSTRIP = 10
CH = 32
NCH = SE // CH
NB = 8
CHH = 32
NCHH = SE // CHH
C2 = 16
RC2 = 32
C3 = 16
C5 = 64


def _rsqrt_newton(x):
    i = lax.bitcast_convert_type(x, jnp.int32)
    i = jnp.full_like(i, 0x5F3759DF) - lax.shift_right_arithmetic(
        i, jnp.ones_like(i))
    y = lax.bitcast_convert_type(i, jnp.float32)
    half_x = x * 0.5
    for _ in range(4):
        y = y * (1.5 - half_x * y * y)
    return y


def _sc_body(rows_hbm, cols_hbm, h_hbm, bias_hbm,
             item_hbm, hs_hbm,
             rstage, cstage, gbuf, zbuf, dinv_t, biasv,
             hist_s, agg_s,
             sem0, semg, sems):
    cid = lax.axis_index("c")
    sid = lax.axis_index("s")
    e0 = sid * EC
    r0t = sid * RT
    hs0 = cid * N2
    real = jnp.minimum(N - r0t, RT)
    iota = lax.iota(jnp.int32, L)

    pltpu.sync_copy(bias_hbm.at[pl.ds(cid * W, W)], biasv)
    scope = jax.named_scope

    def _fill_zbuf(val):
        def _row(i, carry):
            zbuf[i, pl.ds(0, L)] = jnp.full((L,), val, jnp.float32)
            return carry
        lax.fori_loop(0, RC2, _row, 0)

    _fill_zbuf(0.0)

    def _zero_chunk(j, carry):
        pltpu.sync_copy(zbuf.at[pl.ds(0, C2)],
                        hist_s.at[pl.ds(r0t + j * C2, C2)])
        return carry
    lax.fori_loop(0, real // C2, _zero_chunk, 0)

    @pl.when(sid == 0)
    def _zero_hist_tail():
        pltpu.sync_copy(zbuf.at[pl.ds(0, HROWS - N)],
                        hist_s.at[pl.ds(N, HROWS - N)])

    plsc.subcore_barrier()
    _fill_zbuf(1.0)
    p1 = scope("p1_hist"); p1.__enter__()

    for s in range(NSTRIP):
        nreal = min(EC - s * SE, SE)
        pltpu.sync_copy(rows_hbm.at[pl.ds(e0 + s * SE, nreal)],
                        rstage.at[pl.ds(0, nreal)])
        for i in range((SE - nreal) // L):
            rstage[pl.ds(nreal + i * L, L)] = jnp.full((L,), SENT, jnp.int32)

        def _fire(j, carry):
            pltpu.async_copy(zbuf, hist_s.at[rstage.at[pl.ds(j * CHH, CHH)]],
                             sem0, add=True)
            return carry
        lax.fori_loop(0, NCHH, _fire, 0)

        def _drain(j, carry):
            pltpu.make_async_copy(
                zbuf, hist_s.at[rstage.at[pl.ds(0, CHH)]], sem0).wait()
            return carry
        lax.fori_loop(0, NCHH, _drain, 0)

    p1.__exit__(None, None, None)
    plsc.subcore_barrier()
    p2 = scope("p2_dinv"); p2.__enter__()

    def _deg_chunk(c, carry):
        s0 = jnp.minimum(c * RC2, real - RC2)
        pltpu.sync_copy(hist_s.at[pl.ds(r0t + s0, RC2)], zbuf)

        def _deg_group(g, carry2):
            acc = jnp.zeros((L,), jnp.float32)
            for i in range(L):
                acc = jnp.where(iota == i, zbuf[g * L + i, pl.ds(0, L)], acc)
            dinv_t[pl.ds(s0 + g * L, L)] = _rsqrt_newton(acc + 1.0)
            return carry2
        lax.fori_loop(0, RC2 // L, _deg_group, 0)
        return carry
    lax.fori_loop(0, (real + RC2 - 1) // RC2, _deg_chunk, 0)

    p2.__exit__(None, None, None)
    p3 = scope("p3_hs"); p3.__enter__()

    nhs = jnp.minimum(N - r0t, RT) // C3

    def _hs_chunk(c, carry):
        r0 = r0t + c * C3
        pltpu.sync_copy(h_hbm.at[pl.ds(cid * N + r0, C3)], gbuf.at[pl.ds(0, C3)])
        dvs = dinv_t[pl.ds(c * C3, L)]
        for i in range(L):
            dv = jnp.full((L,), dvs[i])
            for k in range(W // L):
                gbuf[i, pl.ds(k * L, L)] = gbuf[i, pl.ds(k * L, L)] * dv
        pltpu.sync_copy(gbuf.at[pl.ds(0, C3)],
                        hs_hbm.at[pl.ds(hs0 + r0, C3)])
        pltpu.sync_copy(gbuf.at[pl.ds(0, C3)], agg_s.at[pl.ds(r0, C3)])
        return carry
    lax.fori_loop(0, nhs, _hs_chunk, 0)

    p3.__exit__(None, None, None)
    plsc.subcore_barrier()
    p4 = scope("p4_edges"); p4.__enter__()

    bufs = [gbuf.at[pl.ds(b * CH, CH)] for b in range(NB)]
    for s in range(NSTRIP):
        nreal = min(EC - s * SE, SE)
        pltpu.sync_copy(rows_hbm.at[pl.ds(e0 + s * SE, nreal)],
                        rstage.at[pl.ds(0, nreal)])
        pltpu.sync_copy(cols_hbm.at[pl.ds(e0 + s * SE, nreal)],
                        cstage.at[pl.ds(0, nreal)])
        for i in range((SE - nreal) // L):
            rstage[pl.ds(nreal + i * L, L)] = jnp.full((L,), TRASH, jnp.int32)
            cstage[pl.ds(nreal + i * L, L)] = jnp.zeros((L,), jnp.int32)

        def _coff(j, carry):
            cstage[pl.ds(j * L, L)] = cstage[pl.ds(j * L, L)] + hs0
            return carry
        lax.fori_loop(0, SE // L, _coff, 0)

        for b in range(NB):
            pltpu.async_copy(hs_hbm.at[cstage.at[pl.ds(b * CH, CH)]],
                             bufs[b], semg.at[b])

        def _round(r, carry):
            c0 = r * NB
            for b in range(NB):
                pltpu.make_async_copy(hs_hbm.at[pl.ds(0, CH)], bufs[b],
                                      semg.at[b]).wait()
                pltpu.async_copy(
                    bufs[b], agg_s.at[rstage.at[pl.ds((c0 + b) * CH, CH)]],
                    sems.at[b], add=True)
            for b in range(NB):
                pltpu.make_async_copy(
                    bufs[b], agg_s.at[pl.ds(0, CH)], sems.at[b]).wait()

                @pl.when(c0 + NB + b < NCH)
                def _refill(b=b, c0=c0):
                    pltpu.async_copy(
                        hs_hbm.at[cstage.at[pl.ds((c0 + NB + b) * CH, CH)]],
                        bufs[b], semg.at[b])
            return carry
        lax.fori_loop(0, NCH // NB, _round, 0)

    p4.__exit__(None, None, None)
    plsc.subcore_barrier()
    p5 = scope("p5_fin"); p5.__enter__()

    real = jnp.minimum(N - r0t, RT)

    def _fin_chunk(c, carry):
        loff = jnp.minimum(c * C5, real - C5)
        pltpu.sync_copy(agg_s.at[pl.ds(r0t + loff, C5)], gbuf.at[pl.ds(0, C5)])

        def _fin_group(g, carry2):
            dvs = dinv_t[pl.ds(loff + g * L, L)]
            for i in range(L):
                dv = jnp.full((L,), dvs[i])
                row = g * L + i
                for k in range(W // L):
                    v = gbuf[row, pl.ds(k * L, L)] * dv + biasv[pl.ds(k * L, L)]
                    gbuf[row, pl.ds(k * L, L)] = jnp.maximum(v, 0.0)
            return carry2
        lax.fori_loop(0, C5 // L, _fin_group, 0)
        pltpu.sync_copy(gbuf.at[pl.ds(0, C5)],
                        item_hbm.at[pl.ds(cid * N + r0t + loff, C5)])
        return carry
    nfin = (real + C5 - 1) // C5
    lax.fori_loop(0, nfin, _fin_chunk, 0)
    p5.__exit__(None, None, None)


@jax.jit
def _igcn_sc(rows, cols, h, bias):
    mesh = plsc.VectorSubcoreMesh(core_axis_name="c", subcore_axis_name="s",
                                  num_cores=NC, num_subcores=NS)
    f = pl.kernel(
        _sc_body,
        out_type=(
            jax.ShapeDtypeStruct((NC * N, W), jnp.float32),
            jax.ShapeDtypeStruct((NC * N2, W), jnp.float32),
        ),
        mesh=mesh,
        scratch_types=[
            pltpu.VMEM((SE,), jnp.int32),
            pltpu.VMEM((SE,), jnp.int32),
            pltpu.VMEM((NB * CH, W), jnp.float32),
            pltpu.VMEM((RC2, L), jnp.float32),
            pltpu.VMEM((RT,), jnp.float32),
            pltpu.VMEM((W,), jnp.float32),
            pltpu.VMEM_SHARED((HROWS, L), jnp.float32),
            pltpu.VMEM_SHARED((AGGR, W), jnp.float32),
            pltpu.SemaphoreType.DMA,
            pltpu.SemaphoreType.DMA((NB,)),
            pltpu.SemaphoreType.DMA((NB,)),
        ],
    )
    item_h, _hs = f(rows, cols, h, bias)
    return item_h


def kernel(edge_index, user_embeddings, gcn_kernel, gcn_bias):
    rows = edge_index[0].astype(jnp.int32)
    cols = edge_index[1].astype(jnp.int32)
    h2 = gcn_kernel.reshape(N, NC, W).transpose(1, 0, 2).reshape(NC * N, W)
    item_flat = _igcn_sc(rows, cols, h2, gcn_bias)
    item_h = item_flat.reshape(NC, N, W).transpose(1, 0, 2).reshape(N, EMB)
    return (user_embeddings, item_h)


# speedup vs baseline: 1.6737x; 1.0475x over previous
import jax
import jax.numpy as jnp
from jax import lax
from jax.experimental import pallas as pl
from jax.experimental.pallas import tpu as pltpu
from jax.experimental.pallas import tpu_sc as plsc

N = 10000
EMB = 256
E = 160000
NC = 2
NS = 16
L = 16
W = 128

N2 = 10240
RT = 640
AGGR = 10008
TRASH = 10000
HROWS = 10008
SENT = 10000

EC = E // NS
SE = 1024
NSTRIP = 10
CH = 32
NCH = SE // CH
NB = 8
CHH = 32
NCHH = SE // CHH
C2 = 16
RC2 = 32
C3 = 64
C5 = 64


def _rsqrt_newton(x):
    i = lax.bitcast_convert_type(x, jnp.int32)
    i = jnp.full_like(i, 0x5F3759DF) - lax.shift_right_arithmetic(
        i, jnp.ones_like(i))
    y = lax.bitcast_convert_type(i, jnp.float32)
    half_x = x * 0.5
    for _ in range(4):
        y = y * (1.5 - half_x * y * y)
    return y


def _sc_body(rows_hbm, cols_hbm, h_hbm, bias_hbm,
             item_hbm, hs_hbm,
             rstage, cstage, gbuf, zbuf, dinv_t, biasv,
             hist_s, agg_s,
             sem0, semg, sems):
    cid = lax.axis_index("c")
    sid = lax.axis_index("s")
    e0 = sid * EC
    r0t = sid * RT
    hs0 = cid * N2
    real = jnp.minimum(N - r0t, RT)
    iota = lax.iota(jnp.int32, L)

    pltpu.sync_copy(bias_hbm.at[pl.ds(cid * W, W)], biasv)
    scope = jax.named_scope

    def _fill_zbuf(val):
        def _row(i, carry):
            zbuf[i, pl.ds(0, L)] = jnp.full((L,), val, jnp.float32)
            return carry
        lax.fori_loop(0, RC2, _row, 0)

    _fill_zbuf(0.0)

    def _zero_chunk(j, carry):
        pltpu.sync_copy(zbuf.at[pl.ds(0, C2)],
                        hist_s.at[pl.ds(r0t + j * C2, C2)])
        return carry
    lax.fori_loop(0, real // C2, _zero_chunk, 0)

    @pl.when(sid == 0)
    def _zero_hist_tail():
        pltpu.sync_copy(zbuf.at[pl.ds(0, HROWS - N)],
                        hist_s.at[pl.ds(N, HROWS - N)])

    plsc.subcore_barrier()
    _fill_zbuf(1.0)
    p1 = scope("p1_hist"); p1.__enter__()

    for s in range(NSTRIP):
        nreal = min(EC - s * SE, SE)
        pltpu.sync_copy(rows_hbm.at[pl.ds(e0 + s * SE, nreal)],
                        rstage.at[pl.ds(0, nreal)])
        for i in range((SE - nreal) // L):
            rstage[pl.ds(nreal + i * L, L)] = jnp.full((L,), SENT, jnp.int32)

        def _fire(j, carry):
            pltpu.async_copy(zbuf, hist_s.at[rstage.at[pl.ds(j * CHH, CHH)]],
                             sem0, add=True)
            return carry
        lax.fori_loop(0, NCHH, _fire, 0)

        def _drain(j, carry):
            pltpu.make_async_copy(
                zbuf, hist_s.at[rstage.at[pl.ds(0, CHH)]], sem0).wait()
            return carry
        lax.fori_loop(0, NCHH, _drain, 0)

    p1.__exit__(None, None, None)
    plsc.subcore_barrier()
    p2 = scope("p2_dinv"); p2.__enter__()

    def _deg_chunk(c, carry):
        s0 = jnp.minimum(c * RC2, real - RC2)
        pltpu.sync_copy(hist_s.at[pl.ds(r0t + s0, RC2)], zbuf)

        def _deg_group(g, carry2):
            acc = jnp.zeros((L,), jnp.float32)
            for i in range(L):
                acc = jnp.where(iota == i, zbuf[g * L + i, pl.ds(0, L)], acc)
            dinv_t[pl.ds(s0 + g * L, L)] = _rsqrt_newton(acc + 1.0)
            return carry2
        lax.fori_loop(0, RC2 // L, _deg_group, 0)
        return carry
    lax.fori_loop(0, (real + RC2 - 1) // RC2, _deg_chunk, 0)

    p2.__exit__(None, None, None)
    p3 = scope("p3_hs"); p3.__enter__()

    def _hs_chunk(c, carry):
        loff = jnp.minimum(c * C3, real - C3)
        r0 = r0t + loff
        pltpu.sync_copy(h_hbm.at[pl.ds(cid * N + r0, C3)], gbuf.at[pl.ds(0, C3)])

        def _hs_group(g, carry2):
            dvs = dinv_t[pl.ds(loff + g * L, L)]
            for i in range(L):
                dv = jnp.full((L,), dvs[i])
                row = g * L + i
                for k in range(W // L):
                    gbuf[row, pl.ds(k * L, L)] = gbuf[row, pl.ds(k * L, L)] * dv
            return carry2
        lax.fori_loop(0, C3 // L, _hs_group, 0)
        pltpu.sync_copy(gbuf.at[pl.ds(0, C3)],
                        hs_hbm.at[pl.ds(hs0 + r0, C3)])
        pltpu.sync_copy(gbuf.at[pl.ds(0, C3)], agg_s.at[pl.ds(r0, C3)])
        return carry
    lax.fori_loop(0, (real + C3 - 1) // C3, _hs_chunk, 0)

    p3.__exit__(None, None, None)
    plsc.subcore_barrier()
    p4 = scope("p4_edges"); p4.__enter__()

    bufs = [gbuf.at[pl.ds(b * CH, CH)] for b in range(NB)]
    for s in range(NSTRIP):
        nreal = min(EC - s * SE, SE)
        pltpu.sync_copy(rows_hbm.at[pl.ds(e0 + s * SE, nreal)],
                        rstage.at[pl.ds(0, nreal)])
        pltpu.sync_copy(cols_hbm.at[pl.ds(e0 + s * SE, nreal)],
                        cstage.at[pl.ds(0, nreal)])
        for i in range((SE - nreal) // L):
            rstage[pl.ds(nreal + i * L, L)] = jnp.full((L,), TRASH, jnp.int32)
            cstage[pl.ds(nreal + i * L, L)] = jnp.zeros((L,), jnp.int32)

        def _coff(j, carry):
            cstage[pl.ds(j * L, L)] = cstage[pl.ds(j * L, L)] + hs0
            return carry
        lax.fori_loop(0, SE // L, _coff, 0)

        for b in range(NB):
            pltpu.async_copy(hs_hbm.at[cstage.at[pl.ds(b * CH, CH)]],
                             bufs[b], semg.at[b])

        def _round(r, carry):
            c0 = r * NB
            for b in range(NB):
                pltpu.make_async_copy(hs_hbm.at[pl.ds(0, CH)], bufs[b],
                                      semg.at[b]).wait()
                pltpu.async_copy(
                    bufs[b], agg_s.at[rstage.at[pl.ds((c0 + b) * CH, CH)]],
                    sems.at[b], add=True)
            for b in range(NB):
                pltpu.make_async_copy(
                    bufs[b], agg_s.at[pl.ds(0, CH)], sems.at[b]).wait()

                @pl.when(c0 + NB + b < NCH)
                def _refill(b=b, c0=c0):
                    pltpu.async_copy(
                        hs_hbm.at[cstage.at[pl.ds((c0 + NB + b) * CH, CH)]],
                        bufs[b], semg.at[b])
            return carry
        lax.fori_loop(0, NCH // NB, _round, 0)

    p4.__exit__(None, None, None)
    plsc.subcore_barrier()
    p5 = scope("p5_fin"); p5.__enter__()

    real = jnp.minimum(N - r0t, RT)

    def _fin_chunk(c, carry):
        loff = jnp.minimum(c * C5, real - C5)
        pltpu.sync_copy(agg_s.at[pl.ds(r0t + loff, C5)], gbuf.at[pl.ds(0, C5)])

        def _fin_group(g, carry2):
            dvs = dinv_t[pl.ds(loff + g * L, L)]
            for i in range(L):
                dv = jnp.full((L,), dvs[i])
                row = g * L + i
                for k in range(W // L):
                    v = gbuf[row, pl.ds(k * L, L)] * dv + biasv[pl.ds(k * L, L)]
                    gbuf[row, pl.ds(k * L, L)] = jnp.maximum(v, 0.0)
            return carry2
        lax.fori_loop(0, C5 // L, _fin_group, 0)
        pltpu.sync_copy(gbuf.at[pl.ds(0, C5)],
                        item_hbm.at[pl.ds(cid * N + r0t + loff, C5)])
        return carry
    nfin = (real + C5 - 1) // C5
    lax.fori_loop(0, nfin, _fin_chunk, 0)
    p5.__exit__(None, None, None)


@jax.jit
def _igcn_sc(rows, cols, h, bias):
    mesh = plsc.VectorSubcoreMesh(core_axis_name="c", subcore_axis_name="s",
                                  num_cores=NC, num_subcores=NS)
    f = pl.kernel(
        _sc_body,
        out_type=(
            jax.ShapeDtypeStruct((NC * N, W), jnp.float32),
            jax.ShapeDtypeStruct((NC * N2, W), jnp.float32),
        ),
        mesh=mesh,
        scratch_types=[
            pltpu.VMEM((SE,), jnp.int32),
            pltpu.VMEM((SE,), jnp.int32),
            pltpu.VMEM((NB * CH, W), jnp.float32),
            pltpu.VMEM((RC2, L), jnp.float32),
            pltpu.VMEM((RT,), jnp.float32),
            pltpu.VMEM((W,), jnp.float32),
            pltpu.VMEM_SHARED((HROWS, L), jnp.float32),
            pltpu.VMEM_SHARED((AGGR, W), jnp.float32),
            pltpu.SemaphoreType.DMA,
            pltpu.SemaphoreType.DMA((NB,)),
            pltpu.SemaphoreType.DMA((NB,)),
        ],
    )
    item_h, _hs = f(rows, cols, h, bias)
    return item_h


def kernel(edge_index, user_embeddings, gcn_kernel, gcn_bias):
    rows = edge_index[0].astype(jnp.int32)
    cols = edge_index[1].astype(jnp.int32)
    h2 = gcn_kernel.reshape(N, NC, W).transpose(1, 0, 2).reshape(NC * N, W)
    item_flat = _igcn_sc(rows, cols, h2, gcn_bias)
    item_h = item_flat.reshape(NC, N, W).transpose(1, 0, 2).reshape(N, EMB)
    return (user_embeddings, item_h)


# speedup vs baseline: 1.7115x; 1.0225x over previous
import jax
import jax.numpy as jnp
from jax import lax
from jax.experimental import pallas as pl
from jax.experimental.pallas import tpu as pltpu
from jax.experimental.pallas import tpu_sc as plsc

N = 10000
EMB = 256
E = 160000
NC = 2
NS = 16
L = 16
W = 128

N2 = 10240
RT = 640
AGGR = 10008
TRASH = 10000
HROWS = 10008
SENT = 10000

EC = E // NS
SE = 2048
NSTRIP = 5
CH = 32
NCH = SE // CH
NB = 8
CHH = 16
NCHH = SE // CHH
C2 = 16
RC2 = 16
C3 = 64
C5 = 64


def _rsqrt_newton(x):
    i = lax.bitcast_convert_type(x, jnp.int32)
    i = jnp.full_like(i, 0x5F3759DF) - lax.shift_right_arithmetic(
        i, jnp.ones_like(i))
    y = lax.bitcast_convert_type(i, jnp.float32)
    half_x = x * 0.5
    for _ in range(4):
        y = y * (1.5 - half_x * y * y)
    return y


def _sc_body(rows_hbm, cols_hbm, h_hbm, bias_hbm,
             item_hbm, hs_hbm,
             rstage, cstage, gbuf, zbuf, dinv_t, biasv,
             hist_s, agg_s,
             sem0, semg, sems):
    cid = lax.axis_index("c")
    sid = lax.axis_index("s")
    e0 = sid * EC
    r0t = sid * RT
    hs0 = cid * N2
    real = jnp.minimum(N - r0t, RT)
    iota = lax.iota(jnp.int32, L)

    pltpu.sync_copy(bias_hbm.at[pl.ds(cid * W, W)], biasv)
    scope = jax.named_scope

    def _fill_zbuf(val):
        def _row(i, carry):
            zbuf[i, pl.ds(0, L)] = jnp.full((L,), val, jnp.float32)
            return carry
        lax.fori_loop(0, RC2, _row, 0)

    _fill_zbuf(0.0)

    def _zero_chunk(j, carry):
        pltpu.sync_copy(zbuf.at[pl.ds(0, C2)],
                        hist_s.at[pl.ds(r0t + j * C2, C2)])
        return carry
    lax.fori_loop(0, real // C2, _zero_chunk, 0)

    @pl.when(sid == 0)
    def _zero_hist_tail():
        pltpu.sync_copy(zbuf.at[pl.ds(0, HROWS - N)],
                        hist_s.at[pl.ds(N, HROWS - N)])

    plsc.subcore_barrier()
    _fill_zbuf(1.0)
    p1 = scope("p1_hist"); p1.__enter__()

    for s in range(NSTRIP):
        nreal = min(EC - s * SE, SE)
        pltpu.sync_copy(rows_hbm.at[pl.ds(e0 + s * SE, nreal)],
                        rstage.at[pl.ds(0, nreal)])
        for i in range((SE - nreal) // L):
            rstage[pl.ds(nreal + i * L, L)] = jnp.full((L,), SENT, jnp.int32)

        def _fire(j, carry):
            pltpu.async_copy(zbuf, hist_s.at[rstage.at[pl.ds(j * CHH, CHH)]],
                             sem0, add=True)
            return carry
        lax.fori_loop(0, NCHH, _fire, 0)

        def _drain(j, carry):
            pltpu.make_async_copy(
                zbuf, hist_s.at[rstage.at[pl.ds(0, CHH)]], sem0).wait()
            return carry
        lax.fori_loop(0, NCHH, _drain, 0)

    p1.__exit__(None, None, None)
    plsc.subcore_barrier()
    p2 = scope("p2_dinv"); p2.__enter__()

    def _deg_chunk(c, carry):
        s0 = jnp.minimum(c * RC2, real - RC2)
        pltpu.sync_copy(hist_s.at[pl.ds(r0t + s0, RC2)], zbuf)

        def _deg_group(g, carry2):
            acc = jnp.zeros((L,), jnp.float32)
            for i in range(L):
                acc = jnp.where(iota == i, zbuf[g * L + i, pl.ds(0, L)], acc)
            dinv_t[pl.ds(s0 + g * L, L)] = _rsqrt_newton(acc + 1.0)
            return carry2
        lax.fori_loop(0, RC2 // L, _deg_group, 0)
        return carry
    lax.fori_loop(0, (real + RC2 - 1) // RC2, _deg_chunk, 0)

    p2.__exit__(None, None, None)
    p3 = scope("p3_hs"); p3.__enter__()

    def _hs_chunk(c, carry):
        loff = jnp.minimum(c * C3, real - C3)
        r0 = r0t + loff
        pltpu.sync_copy(h_hbm.at[pl.ds(cid * N + r0, C3)], gbuf.at[pl.ds(0, C3)])

        def _hs_group(g, carry2):
            dvs = dinv_t[pl.ds(loff + g * L, L)]
            for i in range(L):
                dv = jnp.full((L,), dvs[i])
                row = g * L + i
                for k in range(W // L):
                    gbuf[row, pl.ds(k * L, L)] = gbuf[row, pl.ds(k * L, L)] * dv
            return carry2
        lax.fori_loop(0, C3 // L, _hs_group, 0)
        pltpu.sync_copy(gbuf.at[pl.ds(0, C3)],
                        hs_hbm.at[pl.ds(hs0 + r0, C3)])
        pltpu.sync_copy(gbuf.at[pl.ds(0, C3)], agg_s.at[pl.ds(r0, C3)])
        return carry
    lax.fori_loop(0, (real + C3 - 1) // C3, _hs_chunk, 0)

    p3.__exit__(None, None, None)
    plsc.subcore_barrier()
    p4 = scope("p4_edges"); p4.__enter__()

    bufs = [gbuf.at[pl.ds(b * CH, CH)] for b in range(NB)]
    for s in range(NSTRIP):
        nreal = min(EC - s * SE, SE)
        pltpu.sync_copy(rows_hbm.at[pl.ds(e0 + s * SE, nreal)],
                        rstage.at[pl.ds(0, nreal)])
        pltpu.sync_copy(cols_hbm.at[pl.ds(e0 + s * SE, nreal)],
                        cstage.at[pl.ds(0, nreal)])
        for i in range((SE - nreal) // L):
            rstage[pl.ds(nreal + i * L, L)] = jnp.full((L,), TRASH, jnp.int32)
            cstage[pl.ds(nreal + i * L, L)] = jnp.zeros((L,), jnp.int32)

        def _coff(j, carry):
            cstage[pl.ds(j * L, L)] = cstage[pl.ds(j * L, L)] + hs0
            return carry
        lax.fori_loop(0, SE // L, _coff, 0)

        for b in range(NB):
            pltpu.async_copy(hs_hbm.at[cstage.at[pl.ds(b * CH, CH)]],
                             bufs[b], semg.at[b])

        def _round(r, carry):
            c0 = r * NB
            for b in range(NB):
                pltpu.make_async_copy(hs_hbm.at[pl.ds(0, CH)], bufs[b],
                                      semg.at[b]).wait()
                pltpu.async_copy(
                    bufs[b], agg_s.at[rstage.at[pl.ds((c0 + b) * CH, CH)]],
                    sems.at[b], add=True)
            for b in range(NB):
                pltpu.make_async_copy(
                    bufs[b], agg_s.at[pl.ds(0, CH)], sems.at[b]).wait()

                @pl.when(c0 + NB + b < NCH)
                def _refill(b=b, c0=c0):
                    pltpu.async_copy(
                        hs_hbm.at[cstage.at[pl.ds((c0 + NB + b) * CH, CH)]],
                        bufs[b], semg.at[b])
            return carry
        lax.fori_loop(0, NCH // NB, _round, 0)

    p4.__exit__(None, None, None)
    plsc.subcore_barrier()
    p5 = scope("p5_fin"); p5.__enter__()

    real = jnp.minimum(N - r0t, RT)

    def _fin_chunk(c, carry):
        loff = jnp.minimum(c * C5, real - C5)
        pltpu.sync_copy(agg_s.at[pl.ds(r0t + loff, C5)], gbuf.at[pl.ds(0, C5)])

        def _fin_group(g, carry2):
            dvs = dinv_t[pl.ds(loff + g * L, L)]
            for i in range(L):
                dv = jnp.full((L,), dvs[i])
                row = g * L + i
                for k in range(W // L):
                    v = gbuf[row, pl.ds(k * L, L)] * dv + biasv[pl.ds(k * L, L)]
                    gbuf[row, pl.ds(k * L, L)] = jnp.maximum(v, 0.0)
            return carry2
        lax.fori_loop(0, C5 // L, _fin_group, 0)
        pltpu.sync_copy(gbuf.at[pl.ds(0, C5)],
                        item_hbm.at[pl.ds(cid * N + r0t + loff, C5)])
        return carry
    nfin = (real + C5 - 1) // C5
    lax.fori_loop(0, nfin, _fin_chunk, 0)
    p5.__exit__(None, None, None)


@jax.jit
def _igcn_sc(rows, cols, h, bias):
    mesh = plsc.VectorSubcoreMesh(core_axis_name="c", subcore_axis_name="s",
                                  num_cores=NC, num_subcores=NS)
    f = pl.kernel(
        _sc_body,
        out_type=(
            jax.ShapeDtypeStruct((NC * N, W), jnp.float32),
            jax.ShapeDtypeStruct((NC * N2, W), jnp.float32),
        ),
        mesh=mesh,
        scratch_types=[
            pltpu.VMEM((SE,), jnp.int32),
            pltpu.VMEM((SE,), jnp.int32),
            pltpu.VMEM((NB * CH, W), jnp.float32),
            pltpu.VMEM((RC2, L), jnp.float32),
            pltpu.VMEM((RT,), jnp.float32),
            pltpu.VMEM((W,), jnp.float32),
            pltpu.VMEM_SHARED((HROWS, L), jnp.float32),
            pltpu.VMEM_SHARED((AGGR, W), jnp.float32),
            pltpu.SemaphoreType.DMA,
            pltpu.SemaphoreType.DMA((NB,)),
            pltpu.SemaphoreType.DMA((NB,)),
        ],
    )
    item_h, _hs = f(rows, cols, h, bias)
    return item_h


def kernel(edge_index, user_embeddings, gcn_kernel, gcn_bias):
    rows = edge_index[0].astype(jnp.int32)
    cols = edge_index[1].astype(jnp.int32)
    h2 = gcn_kernel.reshape(N, NC, W).transpose(1, 0, 2).reshape(NC * N, W)
    item_flat = _igcn_sc(rows, cols, h2, gcn_bias)
    item_h = item_flat.reshape(NC, N, W).transpose(1, 0, 2).reshape(N, EMB)
    return (user_embeddings, item_h)
